# baseline (speedup 1.0000x reference)
import jax
import jax.numpy as jnp
from jax import lax
from jax.experimental import pallas as pl
from jax.experimental.pallas import tpu as pltpu
from jax.experimental.pallas import tpu_sc as plsc

NC = 2
NS = 16
L = 16
CH = 80
K = 4
KG = 4


def _sc_scatter(x_hbm, comb_hbm, ew_hbm, zeros_hbm, hpart_hbm,
                cbuf, ebuf, rows_v, h_sp, csem, esem, gsem, ssem):
    npad, d = zeros_hbm.shape
    ngb = comb_hbm.shape[1]
    nch = ngb * K
    rpt = npad // NS

    c = lax.axis_index("c")
    s = lax.axis_index("s")
    wid = s * NC + c

    row0 = s * rpt
    pltpu.sync_copy(zeros_hbm.at[pl.ds(row0, rpt)], h_sp.at[pl.ds(row0, rpt)])
    plsc.subcore_barrier()

    def blk_start(g, slot):
        pltpu.async_copy(comb_hbm.at[wid, g], cbuf.at[slot], csem.at[slot])
        pltpu.async_copy(ew_hbm.at[wid, g], ebuf.at[slot], esem.at[slot])

    def blk_wait(g, slot):
        pltpu.make_async_copy(comb_hbm.at[wid, g], cbuf.at[slot],
                              csem.at[slot]).wait()
        pltpu.make_async_copy(ew_hbm.at[wid, g], ebuf.at[slot],
                              esem.at[slot]).wait()

    def src_ref(ci):
        return cbuf.at[(ci >> 2) & (KG - 1), (ci & (K - 1)) * 2]

    def dst_ref(ci):
        return cbuf.at[(ci >> 2) & (KG - 1), (ci & (K - 1)) * 2 + 1]

    def gather_start(ci, b):
        pltpu.async_copy(x_hbm.at[src_ref(ci)], rows_v.at[b], gsem.at[b])

    def gather_wait(ci, b):
        pltpu.make_async_copy(x_hbm.at[src_ref(ci)], rows_v.at[b],
                              gsem.at[b]).wait()

    def scatter_start(ci, b):
        pltpu.async_copy(rows_v.at[b], h_sp.at[dst_ref(ci)],
                         ssem.at[b], add=True)

    def scatter_wait(ci, b):
        pltpu.make_async_copy(rows_v.at[b], h_sp.at[dst_ref(ci)],
                              ssem.at[b]).wait()

    def scale_chunk(ci, b, slot):
        def tgroup(t, carry):
            ewvec = ebuf[slot, b, pl.ds(t * L, L)]
            for k in range(L):
                w = ewvec[k]
                i = t * L + k
                for j in range(d // L):
                    v = rows_v[b, i, pl.ds(j * L, L)]
                    rows_v[b, i, pl.ds(j * L, L)] = v * w
            return carry
        lax.fori_loop(0, CH // L, tgroup, 0)

    blk_start(0, 0)
    blk_start(1, 1)
    blk_wait(0, 0)
    for b in range(K):
        gather_start(b, b)

    def group(g, carry):
        slot = g & (KG - 1)

        @pl.when(g + 2 < ngb)
        def _():
            blk_start(g + 2, (g + 2) & (KG - 1))

        @pl.when(g + 1 < ngb)
        def _():
            blk_wait(g + 1, (g + 1) & (KG - 1))

        for b in range(K):
            ci = g * K + b
            gather_wait(ci, b)
            scale_chunk(ci, b, slot)
            cprev = ci - 1
            bprev = (b - 1) % K

            @pl.when(cprev >= 0)
            def _():
                scatter_wait(cprev, bprev)

                @pl.when(cprev + K < nch)
                def _():
                    gather_start(cprev + K, bprev)

            scatter_start(ci, b)
        return carry

    lax.fori_loop(0, ngb, group, 0)
    scatter_wait(nch - 1, (nch - 1) % K)
    plsc.subcore_barrier()

    pltpu.sync_copy(h_sp.at[pl.ds(row0, rpt)], hpart_hbm.at[c, pl.ds(row0, rpt)])


def _tc_linear(h_ref, w_ref, b_ref, o_ref):
    h = h_ref[0] + h_ref[1]
    o = lax.dot_general(h, w_ref[...], (((1,), (1,)), ((), ())),
                        preferred_element_type=jnp.float32)
    o_ref[...] = o + b_ref[...]


def kernel(x, edge_index, edge_weight, W, b):
    n, d = x.shape
    e = edge_weight.shape[0]
    nw = NC * NS
    epw = e // nw
    blk = K * CH
    epw_p = ((epw + blk - 1) // blk) * blk
    ngb = epw_p // blk
    pad = epw_p - epw

    ei = edge_index.astype(jnp.int32)
    srcp = jnp.pad(ei[0].reshape(nw, epw), ((0, 0), (0, pad)))
    dstp = jnp.pad(ei[1].reshape(nw, epw), ((0, 0), (0, pad)))
    ewp = jnp.pad(edge_weight.reshape(nw, epw), ((0, 0), (0, pad)))
    comb = jnp.stack([srcp.reshape(nw, ngb, K, CH),
                      dstp.reshape(nw, ngb, K, CH)], axis=3)
    comb = comb.reshape(nw, ngb, 2 * K, CH)
    ewb = ewp.reshape(nw, ngb, K, CH)

    npad = ((n + 16 * NS - 1) // (16 * NS)) * (16 * NS)
    zeros = jnp.zeros((npad, d), jnp.float32)

    mesh = plsc.VectorSubcoreMesh(core_axis_name="c", subcore_axis_name="s")
    sc_call = pl.kernel(
        _sc_scatter,
        out_type=jax.ShapeDtypeStruct((NC, npad, d), jnp.float32),
        mesh=mesh,
        scratch_types=[
            pltpu.VMEM((KG, 2 * K, CH), jnp.int32),
            pltpu.VMEM((KG, K, CH), jnp.float32),
            pltpu.VMEM((K, CH, d), jnp.float32),
            pltpu.VMEM_SHARED((npad, d), jnp.float32),
            pltpu.SemaphoreType.DMA((KG,)),
            pltpu.SemaphoreType.DMA((KG,)),
            pltpu.SemaphoreType.DMA((K,)),
            pltpu.SemaphoreType.DMA((K,)),
        ],
    )
    hpart = sc_call(x, comb, ewb, zeros)

    bn = 2000
    out = pl.pallas_call(
        _tc_linear,
        grid=(n // bn,),
        in_specs=[
            pl.BlockSpec((NC, bn, d), lambda i: (0, i, 0)),
            pl.BlockSpec((d, d), lambda i: (0, 0)),
            pl.BlockSpec((1, d), lambda i: (0, 0)),
        ],
        out_specs=pl.BlockSpec((bn, d), lambda i: (i, 0)),
        out_shape=jax.ShapeDtypeStruct((n, d), jnp.float32),
    )(hpart, W, b.reshape(1, d))
    return out


# speedup vs baseline: 2.6111x; 2.6111x over previous
import jax
import jax.numpy as jnp
from jax import lax
from jax.experimental import pallas as pl
from jax.experimental.pallas import tpu as pltpu
from jax.experimental.pallas import tpu_sc as plsc

NC = 2
NS = 16
L = 16
CH = 80
K = 4
KI = 8


def _sc_scatter(x_hbm, comb_hbm, ew_hbm, zeros_hbm, hpart_hbm,
                cbuf, ebuf, rows_v, h_sp, csem, esem, gsem, ssem):
    npad, d = zeros_hbm.shape
    nch = comb_hbm.shape[1]
    rpt = npad // NS

    c = lax.axis_index("c")
    s = lax.axis_index("s")
    wid = s * NC + c

    row0 = s * rpt
    pltpu.sync_copy(zeros_hbm.at[pl.ds(row0, rpt)], h_sp.at[pl.ds(row0, rpt)])
    plsc.subcore_barrier()

    def idx_start(ci, slot):
        pltpu.async_copy(comb_hbm.at[wid, ci], cbuf.at[slot], csem.at[slot])
        pltpu.async_copy(ew_hbm.at[wid, ci], ebuf.at[slot], esem.at[slot])

    def idx_wait(ci, slot):
        pltpu.make_async_copy(comb_hbm.at[wid, ci], cbuf.at[slot],
                              csem.at[slot]).wait()
        pltpu.make_async_copy(ew_hbm.at[wid, ci], ebuf.at[slot],
                              esem.at[slot]).wait()

    def gather_start(ci, b):
        pltpu.async_copy(x_hbm.at[cbuf.at[ci & (KI - 1), 0]], rows_v.at[b],
                         gsem.at[b])

    def gather_wait(ci, b):
        pltpu.make_async_copy(x_hbm.at[cbuf.at[ci & (KI - 1), 0]],
                              rows_v.at[b], gsem.at[b]).wait()

    def scatter_start(ci, b):
        pltpu.async_copy(rows_v.at[b], h_sp.at[cbuf.at[ci & (KI - 1), 1]],
                         ssem.at[b], add=True)

    def scatter_wait(ci, b):
        pltpu.make_async_copy(rows_v.at[b], h_sp.at[cbuf.at[ci & (KI - 1), 1]],
                              ssem.at[b]).wait()

    def scale_chunk(ci, b):
        slot = ci & (KI - 1)

        def tgroup(t, carry):
            ewvec = ebuf[slot, pl.ds(t * L, L)]
            for k in range(L):
                w = ewvec[k]
                i = t * L + k
                for j in range(d // L):
                    v = rows_v[b, i, pl.ds(j * L, L)]
                    rows_v[b, i, pl.ds(j * L, L)] = v * w
            return carry
        lax.fori_loop(0, CH // L, tgroup, 0)

    for b in range(K):
        idx_start(b, b)
    for b in range(K):
        idx_wait(b, b)
        gather_start(b, b)

    def step(ci, b):
        gather_wait(ci, b)
        scale_chunk(ci, b)
        cprev = ci - 1
        bprev = (b - 1) % K

        @pl.when(cprev >= 0)
        def _():
            scatter_wait(cprev, bprev)

            @pl.when(cprev + K < nch)
            def _():
                idx_wait(cprev + K, (cprev + K) & (KI - 1))
                gather_start(cprev + K, bprev)

        scatter_start(ci, b)

        @pl.when(ci + K < nch)
        def _():
            idx_start(ci + K, (ci + K) & (KI - 1))

    def group(g, carry):
        for b in range(K):
            step(g * K + b, b)
        return carry

    ngroups = nch // K
    lax.fori_loop(0, ngroups, group, 0)
    for r in range(ngroups * K, nch):
        step(r, r % K)
    scatter_wait(nch - 1, (nch - 1) % K)
    plsc.subcore_barrier()

    pltpu.sync_copy(h_sp.at[pl.ds(row0, rpt)], hpart_hbm.at[c, pl.ds(row0, rpt)])


def _tc_linear(h_ref, w_ref, b_ref, o_ref):
    h = h_ref[0] + h_ref[1]
    o = lax.dot_general(h, w_ref[...], (((1,), (1,)), ((), ())),
                        preferred_element_type=jnp.float32)
    o_ref[...] = o + b_ref[...]


def kernel(x, edge_index, edge_weight, W, b):
    n, d = x.shape
    e = edge_weight.shape[0]
    nw = NC * NS
    epw = e // nw
    nch = epw // CH
    ei = edge_index.astype(jnp.int32)
    comb = jnp.stack(
        [ei[0].reshape(nw, nch, CH), ei[1].reshape(nw, nch, CH)], axis=2)
    ew3 = edge_weight.reshape(nw, nch, CH)
    npad = ((n + 8 * NS - 1) // (8 * NS)) * (8 * NS)
    zeros = jnp.zeros((npad, d), jnp.float32)

    mesh = plsc.VectorSubcoreMesh(core_axis_name="c", subcore_axis_name="s")
    sc_call = pl.kernel(
        _sc_scatter,
        out_type=jax.ShapeDtypeStruct((NC, npad, d), jnp.float32),
        mesh=mesh,
        scratch_types=[
            pltpu.VMEM((KI, 2, CH), jnp.int32),
            pltpu.VMEM((KI, CH), jnp.float32),
            pltpu.VMEM((K, CH, d), jnp.float32),
            pltpu.VMEM_SHARED((npad, d), jnp.float32),
            pltpu.SemaphoreType.DMA((KI,)),
            pltpu.SemaphoreType.DMA((KI,)),
            pltpu.SemaphoreType.DMA((K,)),
            pltpu.SemaphoreType.DMA((K,)),
        ],
    )
    hpart = sc_call(x, comb, ew3, zeros)

    bn = 1000
    out = pl.pallas_call(
        _tc_linear,
        grid=(n // bn,),
        in_specs=[
            pl.BlockSpec((NC, bn, d), lambda i: (0, i, 0)),
            pl.BlockSpec((d, d), lambda i: (0, 0)),
            pl.BlockSpec((1, d), lambda i: (0, 0)),
        ],
        out_specs=pl.BlockSpec((bn, d), lambda i: (i, 0)),
        out_shape=jax.ShapeDtypeStruct((n, d), jnp.float32),
    )(hpart, W, b.reshape(1, d))
    return out
